# baseline (device time: 84761 ns/iter reference)
import jax
import jax.numpy as jnp
from jax import lax
from jax.experimental import pallas as pl
from jax.experimental.pallas import tpu as pltpu

N_DEV = 4


def kernel(x, router_W, route_idx, expert_W, shared_W):
    n_tok, d = x.shape
    e_per, _, h = expert_W.shape

    def body(x_ref, rw_ref, idx_ref, ew_ref, sw_ref, out_ref,
             comm_ref, send_sems, recv_sems):
        my = lax.axis_index("i")
        left = lax.rem(my - 1 + N_DEV, N_DEV)
        right = lax.rem(my + 1, N_DEV)

        barrier = pltpu.get_barrier_semaphore()
        for nbr in (left, right):
            pl.semaphore_signal(barrier, inc=1, device_id=(nbr,),
                                device_id_type=pl.DeviceIdType.MESH)
        pl.semaphore_wait(barrier, 2)

        comm_ref[0] = ew_ref[...]

        xv = x_ref[...]
        idx = idx_ref[...]
        scores = jnp.dot(xv, rw_ref[...], preferred_element_type=jnp.float32)
        s_max = jnp.max(scores, axis=-1, keepdims=True)
        e_s = jnp.exp(scores - s_max)
        probs = e_s / jnp.sum(e_s, axis=-1, keepdims=True)
        iota = lax.broadcasted_iota(jnp.int32, scores.shape, 1)
        p_sel = jnp.sum(jnp.where(iota == idx, probs, 0.0), axis=-1,
                        keepdims=True)

        acc = jnp.dot(xv, sw_ref[...], preferred_element_type=jnp.float32)

        for hop in range(N_DEV):
            if hop < N_DEV - 1:
                rdma = pltpu.make_async_remote_copy(
                    src_ref=comm_ref.at[hop],
                    dst_ref=comm_ref.at[hop + 1],
                    send_sem=send_sems.at[hop],
                    recv_sem=recv_sems.at[hop],
                    device_id=(right,),
                    device_id_type=pl.DeviceIdType.MESH,
                )
                rdma.start()
            origin = lax.rem(my - hop + N_DEV, N_DEV)
            for j in range(e_per):
                e_id = origin * e_per + j
                sel = jnp.where(idx == e_id, p_sel, 0.0)
                acc = acc + jnp.dot(xv * sel, comm_ref[hop, j],
                                    preferred_element_type=jnp.float32)
            if hop < N_DEV - 1:
                rdma.wait()

        out_ref[...] = acc

    return pl.pallas_call(
        body,
        out_shape=jax.ShapeDtypeStruct((n_tok, h), jnp.float32),
        in_specs=[pl.BlockSpec(memory_space=pltpu.VMEM)] * 5,
        out_specs=pl.BlockSpec(memory_space=pltpu.VMEM),
        scratch_shapes=[
            pltpu.VMEM((N_DEV, e_per, d, h), jnp.float32),
            pltpu.SemaphoreType.DMA((N_DEV - 1,)),
            pltpu.SemaphoreType.DMA((N_DEV - 1,)),
        ],
        compiler_params=pltpu.CompilerParams(collective_id=0),
    )(x, router_W, route_idx, expert_W, shared_W)


# device time: 50647 ns/iter; 1.6736x vs baseline; 1.6736x over previous
import jax
import jax.numpy as jnp
from jax import lax
from jax.experimental import pallas as pl
from jax.experimental.pallas import tpu as pltpu

N_DEV = 4


def kernel(x, router_W, route_idx, expert_W, shared_W):
    n_tok, d = x.shape
    e_per, _, h = expert_W.shape
    half = e_per // 2

    def body(x_ref, rw_ref, idx_ref, ew_ref, sw_ref, out_ref,
             cw_ref, ccw_ref, cw_send, cw_recv, ccw_send, ccw_recv):
        my = lax.axis_index("i")
        left = lax.rem(my - 1 + N_DEV, N_DEV)
        right = lax.rem(my + 1, N_DEV)

        barrier = pltpu.get_barrier_semaphore()
        for nbr in (left, right):
            pl.semaphore_signal(barrier, inc=1, device_id=(nbr,),
                                device_id_type=pl.DeviceIdType.MESH)
        pl.semaphore_wait(barrier, 2)

        cw_ref[0] = ew_ref[:half]
        ccw_ref[0] = ew_ref[half:]

        rdma_cw = [None] * (N_DEV - 1)
        rdma_ccw = [None] * (N_DEV - 1)

        def start_hop(hop):
            rdma_cw[hop] = pltpu.make_async_remote_copy(
                src_ref=cw_ref.at[hop],
                dst_ref=cw_ref.at[hop + 1],
                send_sem=cw_send.at[hop],
                recv_sem=cw_recv.at[hop],
                device_id=(right,),
                device_id_type=pl.DeviceIdType.MESH,
            )
            rdma_cw[hop].start()
            rdma_ccw[hop] = pltpu.make_async_remote_copy(
                src_ref=ccw_ref.at[hop],
                dst_ref=ccw_ref.at[hop + 1],
                send_sem=ccw_send.at[hop],
                recv_sem=ccw_recv.at[hop],
                device_id=(left,),
                device_id_type=pl.DeviceIdType.MESH,
            )
            rdma_ccw[hop].start()

        start_hop(0)

        xv = x_ref[...]
        idx = idx_ref[...]
        scores = jnp.dot(xv, rw_ref[...], preferred_element_type=jnp.float32)
        s_max = jnp.max(scores, axis=-1, keepdims=True)
        e_s = jnp.exp(scores - s_max)
        probs = e_s / jnp.sum(e_s, axis=-1, keepdims=True)
        iota = lax.broadcasted_iota(jnp.int32, scores.shape, 1)
        p_sel = jnp.sum(jnp.where(iota == idx, probs, 0.0), axis=-1,
                        keepdims=True)

        acc = jnp.dot(xv, sw_ref[...], preferred_element_type=jnp.float32)

        def compute_slot(hop, acc):
            o_cw = lax.rem(my - hop + N_DEV, N_DEV)
            o_ccw = lax.rem(my + hop, N_DEV)
            for j in range(half):
                sel = jnp.where(idx == o_cw * e_per + j, p_sel, 0.0)
                acc = acc + jnp.dot(xv * sel, cw_ref[hop, j],
                                    preferred_element_type=jnp.float32)
                sel = jnp.where(idx == o_ccw * e_per + half + j, p_sel, 0.0)
                acc = acc + jnp.dot(xv * sel, ccw_ref[hop, j],
                                    preferred_element_type=jnp.float32)
            return acc

        for hop in range(N_DEV):
            if hop > 0:
                rdma_cw[hop - 1].wait_recv()
                rdma_ccw[hop - 1].wait_recv()
                if hop < N_DEV - 1:
                    start_hop(hop)
            acc = compute_slot(hop, acc)
            if hop > 0:
                rdma_cw[hop - 1].wait_send()
                rdma_ccw[hop - 1].wait_send()

        out_ref[...] = acc

    return pl.pallas_call(
        body,
        out_shape=jax.ShapeDtypeStruct((n_tok, h), jnp.float32),
        in_specs=[pl.BlockSpec(memory_space=pltpu.VMEM)] * 5,
        out_specs=pl.BlockSpec(memory_space=pltpu.VMEM),
        scratch_shapes=[
            pltpu.VMEM((N_DEV, half, d, h), jnp.float32),
            pltpu.VMEM((N_DEV, half, d, h), jnp.float32),
            pltpu.SemaphoreType.DMA((N_DEV - 1,)),
            pltpu.SemaphoreType.DMA((N_DEV - 1,)),
            pltpu.SemaphoreType.DMA((N_DEV - 1,)),
            pltpu.SemaphoreType.DMA((N_DEV - 1,)),
        ],
        compiler_params=pltpu.CompilerParams(collective_id=0),
    )(x, router_W, route_idx, expert_W, shared_W)


# device time: 33823 ns/iter; 2.5060x vs baseline; 1.4974x over previous
import jax
import jax.numpy as jnp
from jax import lax
from jax.experimental import pallas as pl
from jax.experimental.pallas import tpu as pltpu

N_DEV = 4


def kernel(x, router_W, route_idx, expert_W, shared_W):
    n_tok, d = x.shape
    e_per, _, h = expert_W.shape
    half = e_per // 2

    def body(x_ref, rw_ref, idx_ref, ew_ref, sw_ref, out_ref,
             cw_ref, ccw_ref, cw_send, cw_recv, ccw_send, ccw_recv):
        my = lax.axis_index("i")
        left = lax.rem(my - 1 + N_DEV, N_DEV)
        right = lax.rem(my + 1, N_DEV)

        barrier = pltpu.get_barrier_semaphore()
        for nbr in (left, right):
            pl.semaphore_signal(barrier, inc=1, device_id=(nbr,),
                                device_id_type=pl.DeviceIdType.MESH)
        pl.semaphore_wait(barrier, 2)

        cw_ref[0] = ew_ref[:half].astype(jnp.bfloat16)
        ccw_ref[0] = ew_ref[half:].astype(jnp.bfloat16)

        rdma_cw = [None] * (N_DEV - 1)
        rdma_ccw = [None] * (N_DEV - 1)

        def start_hop(hop):
            rdma_cw[hop] = pltpu.make_async_remote_copy(
                src_ref=cw_ref.at[hop],
                dst_ref=cw_ref.at[hop + 1],
                send_sem=cw_send.at[hop],
                recv_sem=cw_recv.at[hop],
                device_id=(right,),
                device_id_type=pl.DeviceIdType.MESH,
            )
            rdma_cw[hop].start()
            rdma_ccw[hop] = pltpu.make_async_remote_copy(
                src_ref=ccw_ref.at[hop],
                dst_ref=ccw_ref.at[hop + 1],
                send_sem=ccw_send.at[hop],
                recv_sem=ccw_recv.at[hop],
                device_id=(left,),
                device_id_type=pl.DeviceIdType.MESH,
            )
            rdma_ccw[hop].start()

        start_hop(0)

        xv = x_ref[...]
        idx = idx_ref[...]
        scores = jnp.dot(xv, rw_ref[...], preferred_element_type=jnp.float32)
        s_max = jnp.max(scores, axis=-1, keepdims=True)
        e_s = jnp.exp(scores - s_max)
        probs = e_s / jnp.sum(e_s, axis=-1, keepdims=True)
        iota = lax.broadcasted_iota(jnp.int32, scores.shape, 1)
        p_sel = jnp.sum(jnp.where(iota == idx, probs, 0.0), axis=-1,
                        keepdims=True)

        acc = jnp.dot(xv, sw_ref[...], preferred_element_type=jnp.float32)


        def compute_slot(hop, acc):
            o_cw = lax.rem(my - hop + N_DEV, N_DEV)
            o_ccw = lax.rem(my + hop, N_DEV)
            for j in range(half):
                sel = jnp.where(idx == o_cw * e_per + j, p_sel, 0.0)
                acc = acc + jnp.dot((xv * sel).astype(jnp.bfloat16),
                                    cw_ref[hop, j],
                                    preferred_element_type=jnp.float32)
                sel = jnp.where(idx == o_ccw * e_per + half + j, p_sel, 0.0)
                acc = acc + jnp.dot((xv * sel).astype(jnp.bfloat16),
                                    ccw_ref[hop, j],
                                    preferred_element_type=jnp.float32)
            return acc

        for hop in range(N_DEV):
            if hop > 0:
                rdma_cw[hop - 1].wait_recv()
                rdma_ccw[hop - 1].wait_recv()
                if hop < N_DEV - 1:
                    start_hop(hop)
            acc = compute_slot(hop, acc)
            if hop > 0:
                rdma_cw[hop - 1].wait_send()
                rdma_ccw[hop - 1].wait_send()

        out_ref[...] = acc

    return pl.pallas_call(
        body,
        out_shape=jax.ShapeDtypeStruct((n_tok, h), jnp.float32),
        in_specs=[pl.BlockSpec(memory_space=pltpu.VMEM)] * 5,
        out_specs=pl.BlockSpec(memory_space=pltpu.VMEM),
        scratch_shapes=[
            pltpu.VMEM((N_DEV, half, d, h), jnp.bfloat16),
            pltpu.VMEM((N_DEV, half, d, h), jnp.bfloat16),
            pltpu.SemaphoreType.DMA((N_DEV - 1,)),
            pltpu.SemaphoreType.DMA((N_DEV - 1,)),
            pltpu.SemaphoreType.DMA((N_DEV - 1,)),
            pltpu.SemaphoreType.DMA((N_DEV - 1,)),
        ],
        compiler_params=pltpu.CompilerParams(collective_id=0),
    )(x, router_W, route_idx, expert_W, shared_W)


# device time: 29937 ns/iter; 2.8313x vs baseline; 1.1298x over previous
import jax
import jax.numpy as jnp
from jax import lax
from jax.experimental import pallas as pl
from jax.experimental.pallas import tpu as pltpu

N_DEV = 4


def kernel(x, router_W, route_idx, expert_W, shared_W):
    n_tok, d = x.shape
    e_per, _, h = expert_W.shape
    half = e_per // 2

    def body(x_ref, rw_ref, idx_ref, ew_ref, sw_ref, out_ref,
             cw_ref, ccw_ref, cw_send, cw_recv, ccw_send, ccw_recv):
        my = lax.axis_index("i")
        left = lax.rem(my - 1 + N_DEV, N_DEV)
        right = lax.rem(my + 1, N_DEV)

        barrier = pltpu.get_barrier_semaphore()
        for nbr in (left, right):
            pl.semaphore_signal(barrier, inc=1, device_id=(nbr,),
                                device_id_type=pl.DeviceIdType.MESH)
        pl.semaphore_wait(barrier, 2)

        cw_ref[0] = ew_ref[:half].astype(jnp.bfloat16)
        ccw_ref[0] = ew_ref[half:].astype(jnp.bfloat16)

        rdma_cw = [[None] * half for _ in range(N_DEV - 1)]
        rdma_ccw = [[None] * half for _ in range(N_DEV - 1)]

        def start(hop, sub):
            rdma_cw[hop][sub] = pltpu.make_async_remote_copy(
                src_ref=cw_ref.at[hop, sub],
                dst_ref=cw_ref.at[hop + 1, sub],
                send_sem=cw_send.at[hop, sub],
                recv_sem=cw_recv.at[hop, sub],
                device_id=(right,),
                device_id_type=pl.DeviceIdType.MESH,
            )
            rdma_cw[hop][sub].start()
            rdma_ccw[hop][sub] = pltpu.make_async_remote_copy(
                src_ref=ccw_ref.at[hop, sub],
                dst_ref=ccw_ref.at[hop + 1, sub],
                send_sem=ccw_send.at[hop, sub],
                recv_sem=ccw_recv.at[hop, sub],
                device_id=(left,),
                device_id_type=pl.DeviceIdType.MESH,
            )
            rdma_ccw[hop][sub].start()

        for sub in range(half):
            start(0, sub)

        xv = x_ref[...]
        idx = idx_ref[...]
        scores = jnp.dot(xv, rw_ref[...], preferred_element_type=jnp.float32)
        s_max = jnp.max(scores, axis=-1, keepdims=True)
        e_s = jnp.exp(scores - s_max)
        probs = e_s / jnp.sum(e_s, axis=-1, keepdims=True)
        iota = lax.broadcasted_iota(jnp.int32, scores.shape, 1)
        p_sel = jnp.sum(jnp.where(iota == idx, probs, 0.0), axis=-1,
                        keepdims=True)

        def expert_mm(acc, e_id, w):
            sel = jnp.where(idx == e_id, p_sel, 0.0)
            y = jnp.dot((xv * sel).astype(jnp.bfloat16), w,
                        preferred_element_type=jnp.float32)
            return y if acc is None else acc + y

        acc = None
        for hop in range(N_DEV):
            o_cw = lax.rem(my - hop + N_DEV, N_DEV)
            o_ccw = lax.rem(my + hop, N_DEV)
            for sub in range(half):
                if hop > 0:
                    rdma_cw[hop - 1][sub].wait_recv()
                    rdma_ccw[hop - 1][sub].wait_recv()
                    if hop < N_DEV - 1:
                        start(hop, sub)
                acc = expert_mm(acc, o_cw * e_per + sub, cw_ref[hop, sub])
                acc = expert_mm(acc, o_ccw * e_per + half + sub,
                                ccw_ref[hop, sub])
            if hop == 1:
                acc = acc + jnp.dot(xv, sw_ref[...],
                                    preferred_element_type=jnp.float32)
            if hop > 0:
                for sub in range(half):
                    rdma_cw[hop - 1][sub].wait_send()
                    rdma_ccw[hop - 1][sub].wait_send()

        out_ref[...] = acc

    return pl.pallas_call(
        body,
        out_shape=jax.ShapeDtypeStruct((n_tok, h), jnp.float32),
        in_specs=[pl.BlockSpec(memory_space=pltpu.VMEM)] * 5,
        out_specs=pl.BlockSpec(memory_space=pltpu.VMEM),
        scratch_shapes=[
            pltpu.VMEM((N_DEV, half, d, h), jnp.bfloat16),
            pltpu.VMEM((N_DEV, half, d, h), jnp.bfloat16),
            pltpu.SemaphoreType.DMA((N_DEV - 1, half)),
            pltpu.SemaphoreType.DMA((N_DEV - 1, half)),
            pltpu.SemaphoreType.DMA((N_DEV - 1, half)),
            pltpu.SemaphoreType.DMA((N_DEV - 1, half)),
        ],
        compiler_params=pltpu.CompilerParams(collective_id=0),
    )(x, router_W, route_idx, expert_W, shared_W)
